# Initial kernel scaffold; baseline (speedup 1.0000x reference)
import jax
import jax.numpy as jnp
from jax.experimental import pallas as pl


def kernel(x, edge_index, batch, W1, b1, W2, b2, W3, b3, Wfc, bfc):
    raise NotImplementedError("write your pallas kernel here")


# speedup vs baseline: 3.0154x; 3.0154x over previous
import functools

import jax
import jax.numpy as jnp
from jax import lax
from jax.experimental import pallas as pl
from jax.experimental.pallas import tpu as pltpu
from jax.experimental.pallas import tpu_sc as plsc

N = 10000
E = 320000
D = 128
NG = 64
NC = 2
NS = 16
NW = NC * NS
CHUNK = 128
EPW = 10240
NCHUNK = EPW // CHUNK
E_PAD = NW * EPW
DUMMY = 112
ROWS_PAD = N + DUMMY
RPS = ROWS_PAD // NS


def _sc_segment_sum(h, src_r, dst_r, zeros_hbm):
    mesh = plsc.VectorSubcoreMesh(core_axis_name="c", subcore_axis_name="s")

    @functools.partial(
        pl.kernel,
        mesh=mesh,
        out_type=jax.ShapeDtypeStruct((NC, ROWS_PAD, D), jnp.float32),
        scratch_types=[
            pltpu.VMEM((NCHUNK, CHUNK), jnp.int32),
            pltpu.VMEM((NCHUNK, CHUNK), jnp.int32),
            pltpu.VMEM((CHUNK, D), jnp.float32),
            pltpu.VMEM_SHARED((ROWS_PAD, D), jnp.float32),
            pltpu.SemaphoreType.DMA,
        ],
    )
    def seg_sum(h_hbm, src_hbm, dst_hbm, z_hbm, out_hbm, src_v, dst_v, rows_v,
                acc, sem):
        cid = lax.axis_index("c")
        sid = lax.axis_index("s")
        wid = cid * NS + sid
        pltpu.sync_copy(z_hbm, acc.at[pl.ds(sid * RPS, RPS)])
        pltpu.sync_copy(src_hbm.at[wid], src_v)
        pltpu.sync_copy(dst_hbm.at[wid], dst_v)
        plsc.subcore_barrier()

        @pl.loop(0, NCHUNK)
        def _(j):
            pltpu.async_copy(h_hbm.at[src_v.at[j]], rows_v, sem).wait()
            pltpu.sync_copy(rows_v, acc.at[dst_v.at[j]], add=True)

        plsc.subcore_barrier()
        pltpu.sync_copy(acc.at[pl.ds(sid * RPS, RPS)],
                        out_hbm.at[cid, pl.ds(sid * RPS, RPS)])

    return seg_sum(h, src_r, dst_r, zeros_hbm)


_BR = 2000


def _layer_body(h_ref, p_ref, w_ref, b_ref, o_ref):
    z = h_ref[...] + p_ref[0] + p_ref[1]
    y = lax.dot_general(z, w_ref[...], (((1,), (1,)), ((), ())),
                        preferred_element_type=jnp.float32)
    o_ref[...] = jnp.maximum(y + b_ref[...], 0.0)


def _tc_layer(h, parts, W, b2):
    return pl.pallas_call(
        _layer_body,
        grid=(N // _BR,),
        in_specs=[
            pl.BlockSpec((_BR, D), lambda i: (i, 0)),
            pl.BlockSpec((NC, _BR, D), lambda i: (0, i, 0)),
            pl.BlockSpec((D, D), lambda i: (0, 0)),
            pl.BlockSpec((1, D), lambda i: (0, 0)),
        ],
        out_specs=pl.BlockSpec((_BR, D), lambda i: (i, 0)),
        out_shape=jax.ShapeDtypeStruct((N, D), jnp.float32),
    )(h, parts, W, b2)


def _pool_body(h_ref, batch_ref, w_ref, b_ref, o_ref):
    ids = lax.broadcasted_iota(jnp.int32, (NG, N), 0)
    mask = (ids == batch_ref[...]).astype(jnp.float32)
    sums = lax.dot_general(mask, h_ref[...], (((1,), (0,)), ((), ())),
                           preferred_element_type=jnp.float32)
    counts = jnp.sum(mask, axis=1, keepdims=True)
    pooled = sums / jnp.maximum(counts, 1.0)
    y = lax.dot_general(pooled, w_ref[...], (((1,), (1,)), ((), ())),
                        preferred_element_type=jnp.float32)
    o_ref[...] = y + b_ref[...]


def _tc_pool_fc(h, batch2, Wfc, bfc2):
    return pl.pallas_call(
        _pool_body,
        out_shape=jax.ShapeDtypeStruct((NG, D), jnp.float32),
    )(h, batch2, Wfc, bfc2)


def kernel(x, edge_index, batch, W1, b1, W2, b2, W3, b3, Wfc, bfc):
    src = edge_index[0].astype(jnp.int32)
    dst = edge_index[1].astype(jnp.int32)
    n_pad = E_PAD - E
    pad_src = jnp.zeros((n_pad,), jnp.int32)
    pad_dst = N + (jnp.arange(n_pad, dtype=jnp.int32) % DUMMY)
    src_r = jnp.concatenate([src, pad_src]).reshape(NW, NCHUNK, CHUNK)
    dst_r = jnp.concatenate([dst, pad_dst]).reshape(NW, NCHUNK, CHUNK)
    zeros_hbm = jnp.zeros((RPS, D), jnp.float32)
    batch2 = batch.astype(jnp.int32).reshape(1, N)

    h = x
    for W, b in ((W1, b1), (W2, b2), (W3, b3)):
        parts = _sc_segment_sum(h, src_r, dst_r, zeros_hbm)
        h = _tc_layer(h, parts, W, b.reshape(1, D))
    return _tc_pool_fc(h, batch2, Wfc, bfc.reshape(1, D))


# speedup vs baseline: 3.0595x; 1.0146x over previous
import functools

import jax
import jax.numpy as jnp
from jax import lax
from jax.experimental import pallas as pl
from jax.experimental.pallas import tpu as pltpu
from jax.experimental.pallas import tpu_sc as plsc

N = 10000
E = 320000
D = 128
NG = 64
NC = 2
NS = 16
NW = NC * NS
CHUNK = 128
EPW = 10240
CB = 8
NSB = 10
NCHUNK = NSB * CB
E_PAD = NW * EPW
KBUF = 2
DUMMY = 112
ROWS_PAD = N + DUMMY
RPS = ROWS_PAD // NS


def _sc_segment_sum(h, src_r, dst_r, zeros_hbm):
    mesh = plsc.VectorSubcoreMesh(core_axis_name="c", subcore_axis_name="s")

    @functools.partial(
        pl.kernel,
        mesh=mesh,
        out_type=jax.ShapeDtypeStruct((NC, ROWS_PAD, D), jnp.float32),
        scratch_types=[
            pltpu.VMEM((CB, CHUNK), jnp.int32),
            pltpu.VMEM((CB, CHUNK), jnp.int32),
            pltpu.VMEM((KBUF, CHUNK, D), jnp.float32),
            pltpu.VMEM_SHARED((ROWS_PAD, D), jnp.float32),
        ] + [pltpu.SemaphoreType.DMA] * (2 * KBUF),
    )
    def seg_sum(h_hbm, src_hbm, dst_hbm, z_hbm, out_hbm, src_v, dst_v, rows_v,
                acc, *sems):
        gsems, ssems = sems[:KBUF], sems[KBUF:]
        cid = lax.axis_index("c")
        sid = lax.axis_index("s")
        wid = cid * NS + sid
        pltpu.sync_copy(z_hbm, acc.at[pl.ds(sid * RPS, RPS)])
        plsc.subcore_barrier()

        @pl.loop(0, NSB)
        def _(s):
            pltpu.sync_copy(src_hbm.at[wid, s], src_v)
            pltpu.sync_copy(dst_hbm.at[wid, s], dst_v)

            @pl.loop(0, CB, step=KBUF)
            def _(j):
                gets = [pltpu.async_copy(h_hbm.at[src_v.at[j + b]],
                                         rows_v.at[b], gsems[b])
                        for b in range(KBUF)]
                puts = []
                for b in range(KBUF):
                    gets[b].wait()
                    puts.append(pltpu.async_copy(rows_v.at[b],
                                                 acc.at[dst_v.at[j + b]],
                                                 ssems[b], add=True))
                for p in puts:
                    p.wait()

        plsc.subcore_barrier()
        pltpu.sync_copy(acc.at[pl.ds(sid * RPS, RPS)],
                        out_hbm.at[cid, pl.ds(sid * RPS, RPS)])

    return seg_sum(h, src_r, dst_r, zeros_hbm)


_BR = 2000


def _layer_body(h_ref, p_ref, w_ref, b_ref, o_ref):
    z = h_ref[...] + p_ref[0] + p_ref[1]
    y = lax.dot_general(z, w_ref[...], (((1,), (1,)), ((), ())),
                        preferred_element_type=jnp.float32)
    o_ref[...] = jnp.maximum(y + b_ref[...], 0.0)


def _tc_layer(h, parts, W, b2):
    return pl.pallas_call(
        _layer_body,
        grid=(N // _BR,),
        in_specs=[
            pl.BlockSpec((_BR, D), lambda i: (i, 0)),
            pl.BlockSpec((NC, _BR, D), lambda i: (0, i, 0)),
            pl.BlockSpec((D, D), lambda i: (0, 0)),
            pl.BlockSpec((1, D), lambda i: (0, 0)),
        ],
        out_specs=pl.BlockSpec((_BR, D), lambda i: (i, 0)),
        out_shape=jax.ShapeDtypeStruct((N, D), jnp.float32),
    )(h, parts, W, b2)


def _pool_body(h_ref, batch_ref, w_ref, b_ref, o_ref):
    ids = lax.broadcasted_iota(jnp.int32, (NG, N), 0)
    mask = (ids == batch_ref[...]).astype(jnp.float32)
    sums = lax.dot_general(mask, h_ref[...], (((1,), (0,)), ((), ())),
                           preferred_element_type=jnp.float32)
    counts = jnp.sum(mask, axis=1, keepdims=True)
    pooled = sums / jnp.maximum(counts, 1.0)
    y = lax.dot_general(pooled, w_ref[...], (((1,), (1,)), ((), ())),
                        preferred_element_type=jnp.float32)
    o_ref[...] = y + b_ref[...]


def _tc_pool_fc(h, batch2, Wfc, bfc2):
    return pl.pallas_call(
        _pool_body,
        out_shape=jax.ShapeDtypeStruct((NG, D), jnp.float32),
    )(h, batch2, Wfc, bfc2)


def kernel(x, edge_index, batch, W1, b1, W2, b2, W3, b3, Wfc, bfc):
    src = edge_index[0].astype(jnp.int32)
    dst = edge_index[1].astype(jnp.int32)
    n_pad = E_PAD - E
    pad_src = jnp.zeros((n_pad,), jnp.int32)
    pad_dst = N + (jnp.arange(n_pad, dtype=jnp.int32) % DUMMY)
    src_r = jnp.concatenate([src, pad_src]).reshape(NW, NSB, CB, CHUNK)
    dst_r = jnp.concatenate([dst, pad_dst]).reshape(NW, NSB, CB, CHUNK)
    zeros_hbm = jnp.zeros((RPS, D), jnp.float32)
    batch2 = batch.astype(jnp.int32).reshape(1, N)

    h = x
    for W, b in ((W1, b1), (W2, b2), (W3, b3)):
        parts = _sc_segment_sum(h, src_r, dst_r, zeros_hbm)
        h = _tc_layer(h, parts, W, b.reshape(1, D))
    return _tc_pool_fc(h, batch2, Wfc, bfc.reshape(1, D))


# speedup vs baseline: 3.3762x; 1.1035x over previous
import functools

import jax
import jax.numpy as jnp
from jax import lax
from jax.experimental import pallas as pl
from jax.experimental.pallas import tpu as pltpu
from jax.experimental.pallas import tpu_sc as plsc

N = 10000
E = 320000
D = 128
NG = 64
NC = 2
NS = 16
NW = NC * NS
CHUNK = 64
EPW = 10240
CB = 16
NSB = 10
NCHUNK = NSB * CB
E_PAD = NW * EPW
KBUF = 4
DUMMY = 112
ROWS_PAD = N + DUMMY
RPS = ROWS_PAD // NS


def _sc_segment_sum(h, src_r, dst_r, zeros_hbm):
    mesh = plsc.VectorSubcoreMesh(core_axis_name="c", subcore_axis_name="s")

    @functools.partial(
        pl.kernel,
        mesh=mesh,
        out_type=jax.ShapeDtypeStruct((NC, ROWS_PAD, D), jnp.float32),
        scratch_types=[
            pltpu.VMEM((CB, CHUNK), jnp.int32),
            pltpu.VMEM((CB, CHUNK), jnp.int32),
            pltpu.VMEM((KBUF, CHUNK, D), jnp.float32),
            pltpu.VMEM_SHARED((ROWS_PAD, D), jnp.float32),
        ] + [pltpu.SemaphoreType.DMA] * (2 * KBUF),
    )
    def seg_sum(h_hbm, src_hbm, dst_hbm, z_hbm, out_hbm, src_v, dst_v, rows_v,
                acc, *sems):
        gsems, ssems = sems[:KBUF], sems[KBUF:]
        cid = lax.axis_index("c")
        sid = lax.axis_index("s")
        wid = cid * NS + sid
        pltpu.sync_copy(z_hbm, acc.at[pl.ds(sid * RPS, RPS)])
        plsc.subcore_barrier()

        @pl.loop(0, NSB)
        def _(s):
            pltpu.sync_copy(src_hbm.at[wid, s], src_v)
            pltpu.sync_copy(dst_hbm.at[wid, s], dst_v)

            @pl.loop(0, CB, step=KBUF)
            def _(j):
                gets = [pltpu.async_copy(h_hbm.at[src_v.at[j + b]],
                                         rows_v.at[b], gsems[b])
                        for b in range(KBUF)]
                puts = []
                for b in range(KBUF):
                    gets[b].wait()
                    puts.append(pltpu.async_copy(rows_v.at[b],
                                                 acc.at[dst_v.at[j + b]],
                                                 ssems[b], add=True))
                for p in puts:
                    p.wait()

        plsc.subcore_barrier()
        pltpu.sync_copy(acc.at[pl.ds(sid * RPS, RPS)],
                        out_hbm.at[cid, pl.ds(sid * RPS, RPS)])

    return seg_sum(h, src_r, dst_r, zeros_hbm)


_BR = 2000


def _layer_body(h_ref, p_ref, w_ref, b_ref, o_ref):
    z = h_ref[...] + p_ref[0] + p_ref[1]
    y = lax.dot_general(z, w_ref[...], (((1,), (1,)), ((), ())),
                        preferred_element_type=jnp.float32)
    o_ref[...] = jnp.maximum(y + b_ref[...], 0.0)


def _tc_layer(h, parts, W, b2):
    return pl.pallas_call(
        _layer_body,
        grid=(N // _BR,),
        in_specs=[
            pl.BlockSpec((_BR, D), lambda i: (i, 0)),
            pl.BlockSpec((NC, _BR, D), lambda i: (0, i, 0)),
            pl.BlockSpec((D, D), lambda i: (0, 0)),
            pl.BlockSpec((1, D), lambda i: (0, 0)),
        ],
        out_specs=pl.BlockSpec((_BR, D), lambda i: (i, 0)),
        out_shape=jax.ShapeDtypeStruct((N, D), jnp.float32),
    )(h, parts, W, b2)


def _pool_body(h_ref, batch_ref, w_ref, b_ref, o_ref):
    ids = lax.broadcasted_iota(jnp.int32, (NG, N), 0)
    mask = (ids == batch_ref[...]).astype(jnp.float32)
    sums = lax.dot_general(mask, h_ref[...], (((1,), (0,)), ((), ())),
                           preferred_element_type=jnp.float32)
    counts = jnp.sum(mask, axis=1, keepdims=True)
    pooled = sums / jnp.maximum(counts, 1.0)
    y = lax.dot_general(pooled, w_ref[...], (((1,), (1,)), ((), ())),
                        preferred_element_type=jnp.float32)
    o_ref[...] = y + b_ref[...]


def _tc_pool_fc(h, batch2, Wfc, bfc2):
    return pl.pallas_call(
        _pool_body,
        out_shape=jax.ShapeDtypeStruct((NG, D), jnp.float32),
    )(h, batch2, Wfc, bfc2)


def kernel(x, edge_index, batch, W1, b1, W2, b2, W3, b3, Wfc, bfc):
    src = edge_index[0].astype(jnp.int32)
    dst = edge_index[1].astype(jnp.int32)
    n_pad = E_PAD - E
    pad_src = jnp.zeros((n_pad,), jnp.int32)
    pad_dst = N + (jnp.arange(n_pad, dtype=jnp.int32) % DUMMY)
    src_r = jnp.concatenate([src, pad_src]).reshape(NW, NSB, CB, CHUNK)
    dst_r = jnp.concatenate([dst, pad_dst]).reshape(NW, NSB, CB, CHUNK)
    zeros_hbm = jnp.zeros((RPS, D), jnp.float32)
    batch2 = batch.astype(jnp.int32).reshape(1, N)

    h = x
    for W, b in ((W1, b1), (W2, b2), (W3, b3)):
        parts = _sc_segment_sum(h, src_r, dst_r, zeros_hbm)
        h = _tc_layer(h, parts, W, b.reshape(1, D))
    return _tc_pool_fc(h, batch2, Wfc, bfc.reshape(1, D))
